# baseline (speedup 1.0000x reference)
import functools

import jax
import jax.numpy as jnp
from jax import lax
from jax.experimental import pallas as pl
from jax.experimental.pallas import tpu as pltpu
from jax.experimental.pallas import tpu_sc as plsc

L = 16
POOL = 16
HW = 512
PR = HW // POOL
NBLK = PR * PR
NCHUNK = NBLK // L
CPR = PR // L
OUT_LEN = NBLK + 1
ROW_PAD = ((OUT_LEN + L - 1) // L) * L
B = 256
OUT_ROWS_PER_TILE = B // 32
OFF_MINOR = 16


def _tc_pool_body(x_ref, m_ref, o_ref):
    x = x_ref[0, 0]
    b = (x > 0.0).astype(jnp.float32)
    r1 = lax.broadcasted_iota(jnp.int32, (PR, HW), 0)
    c1 = lax.broadcasted_iota(jnp.int32, (PR, HW), 1)
    p_left = (c1 // POOL == r1).astype(jnp.float32)
    r2 = lax.broadcasted_iota(jnp.int32, (HW, PR), 0)
    c2 = lax.broadcasted_iota(jnp.int32, (HW, PR), 1)
    p_right = (r2 // POOL == c2).astype(jnp.float32)
    rows = jnp.dot(p_left, b, preferred_element_type=jnp.float32)
    counts = jnp.dot(rows, p_right, preferred_element_type=jnp.float32)
    mask = (counts > 0.5).astype(jnp.float32)
    m_ref[...] = mask.astype(jnp.int32)

    rr = lax.broadcasted_iota(jnp.int32, (PR, PR), 0)
    cc = lax.broadcasted_iota(jnp.int32, (PR, PR), 1)
    strict_lower = (rr > cc).astype(jnp.float32)
    row_tot = jnp.sum(mask, axis=1, keepdims=True)
    pre_rows = jnp.dot(strict_lower, row_tot,
                       preferred_element_type=jnp.float32)
    half0 = jnp.sum(mask[:, :L], axis=1, keepdims=True)
    hh = lax.broadcasted_iota(jnp.int32, (PR, OFF_MINOR), 1)
    offs = (jnp.where(hh <= 1, pre_rows, 0.0)
            + jnp.where(hh == 1, half0, 0.0))
    o_ref[...] = offs.astype(jnp.int32)


_tc_pool = pl.pallas_call(
    _tc_pool_body,
    out_shape=(jax.ShapeDtypeStruct((PR, PR), jnp.int32),
               jax.ShapeDtypeStruct((PR, OFF_MINOR), jnp.int32)),
    grid=(1,),
    in_specs=[pl.BlockSpec((1, 1, HW, HW), lambda i: (0, 0, 0, 0))],
    out_specs=(pl.BlockSpec((PR, PR), lambda i: (0, 0)),
               pl.BlockSpec((PR, OFF_MINOR), lambda i: (0, 0))),
)


_mesh = plsc.VectorSubcoreMesh(core_axis_name="c", subcore_axis_name="s")


@functools.partial(
    pl.kernel,
    out_type=jax.ShapeDtypeStruct((B, OUT_LEN), jnp.int32),
    mesh=_mesh,
    compiler_params=pltpu.CompilerParams(needs_layout_passes=False,
                                         use_tc_tiling_on_sc=False),
    scratch_types=[
        pltpu.VMEM((PR, PR), jnp.int32),
        pltpu.VMEM((PR, OFF_MINOR), jnp.int32),
        pltpu.VMEM((ROW_PAD,), jnp.int32),
        pltpu.SemaphoreType.DMA,
    ],
)
def _sc_compact_broadcast(m_hbm, o_hbm, out_hbm, mv, ov, row_v, sem):
    c = lax.axis_index("c")
    s = lax.axis_index("s")
    lanes = lax.broadcasted_iota(jnp.int32, (L,), 0)

    pltpu.sync_copy(m_hbm, mv)
    pltpu.sync_copy(o_hbm, ov)

    one = jnp.ones((L,), jnp.int32)
    row_v[pl.ds(0, L)] = jnp.where(lanes == 0, 0, one)
    for t in range(1, ROW_PAD // L):
        row_v[pl.ds(t * L, L)] = one

    for t in range(NCHUNK):
        m_vec = mv[t // CPR, pl.ds((t % CPR) * L, L)]
        off_vec = ov[t // CPR, pl.ds(0, L)]
        idx = plsc.cumsum(m_vec) + off_vec[t % CPR]
        vals = lanes + (t * L + 1)
        plsc.store_scatter(row_v, [idx], vals, mask=m_vec > 0)

    base = (s * 2 + c) * OUT_ROWS_PER_TILE
    copies = [
        pltpu.async_copy(row_v.at[pl.ds(0, OUT_LEN)], out_hbm.at[base + i],
                         sem)
        for i in range(OUT_ROWS_PER_TILE)
    ]
    for cp in copies:
        cp.wait()


def kernel(ones_mask):
    mask, offs = _tc_pool(ones_mask)
    return _sc_compact_broadcast(mask, offs)


# speedup vs baseline: 1.0616x; 1.0616x over previous
import functools

import jax
import jax.numpy as jnp
from jax import lax
from jax.experimental import pallas as pl
from jax.experimental.pallas import tpu as pltpu
from jax.experimental.pallas import tpu_sc as plsc

L = 16
POOL = 16
HW = 512
PR = HW // POOL
NBLK = PR * PR
NCHUNK = NBLK // L
CPR = PR // L
OUT_LEN = NBLK + 1
ROW_PAD = ((OUT_LEN + L - 1) // L) * L
B = 256
OUT_ROWS_PER_TILE = B // 32
PK = PR + L


def _tc_pool_body(x_ref, p_ref):
    x = x_ref[0, 0]
    b = (x > 0.0).astype(jnp.float32)
    r1 = lax.broadcasted_iota(jnp.int32, (PR, HW), 0)
    c1 = lax.broadcasted_iota(jnp.int32, (PR, HW), 1)
    p_left = (c1 // POOL == r1).astype(jnp.float32)
    r2 = lax.broadcasted_iota(jnp.int32, (HW, PR), 0)
    c2 = lax.broadcasted_iota(jnp.int32, (HW, PR), 1)
    p_right = (r2 // POOL == c2).astype(jnp.float32)
    rows = jnp.dot(p_left, b, preferred_element_type=jnp.float32)
    counts = jnp.dot(rows, p_right, preferred_element_type=jnp.float32)
    mask = (counts > 0.5).astype(jnp.float32)

    rr = lax.broadcasted_iota(jnp.int32, (PR, PR), 0)
    cc = lax.broadcasted_iota(jnp.int32, (PR, PR), 1)
    strict_lower = (rr > cc).astype(jnp.float32)
    row_tot = jnp.sum(mask, axis=1, keepdims=True)
    pre_rows = jnp.dot(strict_lower, row_tot,
                       preferred_element_type=jnp.float32)
    half0 = jnp.sum(mask[:, :L], axis=1, keepdims=True)

    jj = lax.broadcasted_iota(jnp.int32, (PR, PK), 1)
    mask_wide = jnp.pad(mask, ((0, 0), (0, L)))
    packed = jnp.where(jj < PR, mask_wide, 0.0)
    packed = packed + jnp.where((jj == PR) | (jj == PR + 1), pre_rows, 0.0)
    packed = packed + jnp.where(jj == PR + 1, half0, 0.0)
    p_ref[...] = packed.astype(jnp.int32)


_tc_pool = pl.pallas_call(
    _tc_pool_body,
    out_shape=jax.ShapeDtypeStruct((PR, PK), jnp.int32),
    grid=(1,),
    in_specs=[pl.BlockSpec((1, 1, HW, HW), lambda i: (0, 0, 0, 0))],
    out_specs=pl.BlockSpec((PR, PK), lambda i: (0, 0)),
)


_mesh = plsc.VectorSubcoreMesh(core_axis_name="c", subcore_axis_name="s")


@functools.partial(
    pl.kernel,
    out_type=jax.ShapeDtypeStruct((B, OUT_LEN), jnp.int32),
    mesh=_mesh,
    compiler_params=pltpu.CompilerParams(needs_layout_passes=False,
                                         use_tc_tiling_on_sc=False),
    scratch_types=[
        pltpu.VMEM((PR, PK), jnp.int32),
        pltpu.VMEM((ROW_PAD,), jnp.int32),
        pltpu.VMEM((OUT_ROWS_PER_TILE, OUT_LEN), jnp.int32),
    ],
)
def _sc_compact_broadcast(p_hbm, out_hbm, mv, row_v, rep_v):
    c = lax.axis_index("c")
    s = lax.axis_index("s")
    lanes = lax.broadcasted_iota(jnp.int32, (L,), 0)

    pltpu.sync_copy(p_hbm, mv)

    one = jnp.ones((L,), jnp.int32)
    row_v[pl.ds(0, L)] = jnp.where(lanes == 0, 0, one)
    for t in range(1, ROW_PAD // L):
        row_v[pl.ds(t * L, L)] = one

    for t in range(NCHUNK):
        r = t // CPR
        m_vec = mv[r, pl.ds((t % CPR) * L, L)]
        off_vec = mv[r, pl.ds(PR, L)]
        idx = plsc.cumsum(m_vec) + off_vec[t % CPR]
        vals = lanes + (t * L + 1)
        plsc.store_scatter(row_v, [idx], vals, mask=m_vec > 0)

    for k in range(NBLK // L):
        v = row_v[pl.ds(k * L, L)]
        for i in range(OUT_ROWS_PER_TILE):
            rep_v[i, pl.ds(k * L, L)] = v
    v = row_v[pl.ds(OUT_LEN - L, L)]
    for i in range(OUT_ROWS_PER_TILE):
        rep_v[i, pl.ds(OUT_LEN - L, L)] = v

    base = (s * 2 + c) * OUT_ROWS_PER_TILE
    pltpu.sync_copy(rep_v, out_hbm.at[pl.ds(base, OUT_ROWS_PER_TILE)])


def kernel(ones_mask):
    return _sc_compact_broadcast(_tc_pool(ones_mask))


# speedup vs baseline: 1.1191x; 1.0541x over previous
import functools

import jax
import jax.numpy as jnp
from jax import lax
from jax.experimental import pallas as pl
from jax.experimental.pallas import tpu as pltpu
from jax.experimental.pallas import tpu_sc as plsc

L = 16
POOL = 16
HW = 512
PR = HW // POOL
NBLK = PR * PR
NCHUNK = NBLK // L
CPR = PR // L
OUT_LEN = NBLK + 1
ROW_PAD = ((OUT_LEN + L - 1) // L) * L
B = 256
OUT_ROWS_PER_TILE = B // 32
PK = PR + L


def _tc_pool_body(x_ref, p_ref):
    x = x_ref[0, 0]
    b = (x > 0.0).astype(jnp.float32)
    r1 = lax.broadcasted_iota(jnp.int32, (PR, HW), 0)
    c1 = lax.broadcasted_iota(jnp.int32, (PR, HW), 1)
    p_left = (c1 // POOL == r1).astype(jnp.float32)
    r2 = lax.broadcasted_iota(jnp.int32, (HW, PR), 0)
    c2 = lax.broadcasted_iota(jnp.int32, (HW, PR), 1)
    p_right = (r2 // POOL == c2).astype(jnp.float32)
    rows = jnp.dot(p_left, b, preferred_element_type=jnp.float32)
    counts = jnp.dot(rows, p_right, preferred_element_type=jnp.float32)
    mask = (counts > 0.5).astype(jnp.float32)

    rr = lax.broadcasted_iota(jnp.int32, (PR, PR), 0)
    cc = lax.broadcasted_iota(jnp.int32, (PR, PR), 1)
    strict_lower = (rr > cc).astype(jnp.float32)
    row_tot = jnp.sum(mask, axis=1, keepdims=True)
    pre_rows = jnp.dot(strict_lower, row_tot,
                       preferred_element_type=jnp.float32)
    half0 = jnp.sum(mask[:, :L], axis=1, keepdims=True)

    jj = lax.broadcasted_iota(jnp.int32, (PR, PK), 1)
    mask_wide = jnp.pad(mask, ((0, 0), (0, L)))
    packed = jnp.where(jj < PR, mask_wide, 0.0)
    packed = packed + jnp.where((jj == PR) | (jj == PR + 1), pre_rows, 0.0)
    packed = packed + jnp.where(jj == PR + 1, half0, 0.0)
    p_ref[...] = packed.astype(jnp.int32)


_tc_pool = pl.pallas_call(
    _tc_pool_body,
    out_shape=jax.ShapeDtypeStruct((PR, PK), jnp.int32),
    grid=(1,),
    in_specs=[pl.BlockSpec((1, 1, HW, HW), lambda i: (0, 0, 0, 0))],
    out_specs=pl.BlockSpec((PR, PK), lambda i: (0, 0)),
)


_mesh = plsc.VectorSubcoreMesh(core_axis_name="c", subcore_axis_name="s")


@functools.partial(
    pl.kernel,
    out_type=jax.ShapeDtypeStruct((B, OUT_LEN), jnp.int32),
    mesh=_mesh,
    compiler_params=pltpu.CompilerParams(needs_layout_passes=False,
                                         use_tc_tiling_on_sc=False),
    scratch_types=[
        pltpu.VMEM((PR, PK), jnp.int32),
        pltpu.VMEM((ROW_PAD,), jnp.int32),
        pltpu.VMEM((OUT_ROWS_PER_TILE, OUT_LEN), jnp.int32),
        pltpu.VMEM_SHARED((ROW_PAD,), jnp.int32),
        pltpu.SemaphoreType.DMA,
    ],
)
def _sc_compact_broadcast(p_hbm, out_hbm, mv, row_v, rep_v, shared_row, sem):
    c = lax.axis_index("c")
    s = lax.axis_index("s")
    lanes = lax.broadcasted_iota(jnp.int32, (L,), 0)

    @pl.when(s == 0)
    def _compact():
        pltpu.sync_copy(p_hbm, mv)
        one = jnp.ones((L,), jnp.int32)
        row_v[pl.ds(0, L)] = jnp.where(lanes == 0, 0, one)
        for t in range(1, ROW_PAD // L):
            row_v[pl.ds(t * L, L)] = one
        for t in range(NCHUNK):
            r = t // CPR
            m_vec = mv[r, pl.ds((t % CPR) * L, L)]
            off_vec = mv[r, pl.ds(PR, L)]
            idx = plsc.cumsum(m_vec) + off_vec[t % CPR]
            vals = lanes + (t * L + 1)
            plsc.store_scatter(row_v, [idx], vals, mask=m_vec > 0)
        pltpu.sync_copy(row_v, shared_row)

    plsc.subcore_barrier()

    copies = [
        pltpu.async_copy(shared_row.at[pl.ds(0, OUT_LEN)], rep_v.at[i], sem)
        for i in range(OUT_ROWS_PER_TILE)
    ]
    for cp in copies:
        cp.wait()
    base = (s * 2 + c) * OUT_ROWS_PER_TILE
    pltpu.sync_copy(rep_v, out_hbm.at[pl.ds(base, OUT_ROWS_PER_TILE)])


def kernel(ones_mask):
    return _sc_compact_broadcast(_tc_pool(ones_mask))
